# baseline (speedup 1.0000x reference)
import functools

import jax
import jax.numpy as jnp
from jax import lax
from jax.experimental import pallas as pl
from jax.experimental.pallas import tpu as pltpu
from jax.experimental.pallas import tpu_sc as plsc

N = 10000
E = 320000
D = 128
C = 64

NC = 2
NS = 16
NW = NC * NS
EW = E // NW
K = 64
EWP = 10240
PAD = EWP - EW
NCH = EWP // K
NB = 40
NBLK = NCH // NB
NR = 4
ROWS0 = 624
ZR = 48
TAIL0 = NS * ROWS0
TAIL = N - TAIL0

_F32 = jnp.float32
_I32 = jnp.int32


def _sc_scatter_fn():
    mesh = plsc.VectorSubcoreMesh(
        core_axis_name="c", subcore_axis_name="s", num_cores=NC, num_subcores=NS
    )

    @functools.partial(
        pl.kernel,
        out_type=jax.ShapeDtypeStruct((NC, N, D), _F32),
        mesh=mesh,
        compiler_params=pltpu.CompilerParams(use_tc_tiling_on_sc=False),
        scratch_types=dict(
            src_v=pltpu.VMEM((2, NB, K), _I32),
            dst_v=pltpu.VMEM((2, NB, K), _I32),
            w_v=pltpu.VMEM((2, NB * K), _F32),
            rows=pltpu.VMEM((NR, K, D), _F32),
            acc=pltpu.VMEM_SHARED((N, D), _F32),
            isem=pltpu.SemaphoreType.DMA,
            sg0=pltpu.SemaphoreType.DMA,
            sg1=pltpu.SemaphoreType.DMA,
            sg2=pltpu.SemaphoreType.DMA,
            sg3=pltpu.SemaphoreType.DMA,
            ss0=pltpu.SemaphoreType.DMA,
            ss1=pltpu.SemaphoreType.DMA,
            ss2=pltpu.SemaphoreType.DMA,
            ss3=pltpu.SemaphoreType.DMA,
        ),
    )
    def sc_scatter(x_hbm, src_hbm, dst_hbm, w_hbm, out_hbm,
                   src_v, dst_v, w_v, rows, acc, isem,
                   sg0, sg1, sg2, sg3, ss0, ss1, ss2, ss3):
        gsems = (sg0, sg1, sg2, sg3)
        ssems = (ss0, ss1, ss2, ss3)
        c = lax.axis_index("c")
        s = lax.axis_index("s")
        wid = c * NS + s

        @pl.loop(0, ZR)
        def _(i):
            for j in range(D // 16):
                rows[0, i, pl.ds(j * 16, 16)] = jnp.zeros((16,), _F32)

        zsrc = rows.at[0, pl.ds(0, ZR)]
        row0 = s * ROWS0
        for i in range(ROWS0 // ZR):
            pltpu.sync_copy(zsrc, acc.at[pl.ds(row0 + i * ZR, ZR)])

        @pl.when(s == NS - 1)
        def _():
            pltpu.sync_copy(rows.at[0, pl.ds(0, TAIL)],
                            acc.at[pl.ds(TAIL0, TAIL)])

        plsc.subcore_barrier()

        def start_gather(sl, gg, b):
            pltpu.async_copy(x_hbm.at[src_v.at[sl, gg]], rows.at[b], gsems[b])

        def wait_gather(sl, gg, b):
            pltpu.make_async_copy(
                x_hbm.at[src_v.at[sl, gg]], rows.at[b], gsems[b]).wait()

        def start_scatter(sl, gg, b):
            pltpu.async_copy(rows.at[b], acc.at[dst_v.at[sl, gg]], ssems[b],
                             add=True)

        def drain_scatter(sl, b):
            pltpu.make_async_copy(
                rows.at[b], acc.at[dst_v.at[sl, 0]], ssems[b]).wait()

        def scale(sl, gg, b):
            @pl.loop(0, K // 16)
            def _(t):
                wvec = w_v[sl, pl.ds(gg * K + t * 16, 16)]
                for l in range(16):
                    wb = jnp.full((16,), wvec[l], dtype=_F32)
                    row = t * 16 + l
                    for j in range(D // 16):
                        slc = pl.ds(j * 16, 16)
                        rows[b, row, slc] = rows[b, row, slc] * wb

        pltpu.sync_copy(src_hbm.at[wid, 0], src_v.at[0])
        pltpu.sync_copy(dst_hbm.at[wid, 0], dst_v.at[0])
        pltpu.sync_copy(w_hbm.at[wid, 0], w_v.at[0])

        @pl.loop(0, NBLK)
        def _(blk):
            sl = blk % 2
            nsl = 1 - sl

            @pl.when(blk + 1 < NBLK)
            def _():
                pltpu.async_copy(src_hbm.at[wid, blk + 1], src_v.at[nsl], isem)
                pltpu.async_copy(dst_hbm.at[wid, blk + 1], dst_v.at[nsl], isem)
                pltpu.async_copy(w_hbm.at[wid, blk + 1], w_v.at[nsl], isem)

            start_gather(sl, 0, 0)
            start_gather(sl, 1, 1)

            @pl.loop(0, NB // NR)
            def _(q):
                for b4 in range(NR):
                    gg = q * NR + b4
                    wait_gather(sl, gg, b4)
                    scale(sl, gg, b4)
                    start_scatter(sl, gg, b4)
                    b2 = (b4 + 2) % NR

                    @pl.when(gg >= 2)
                    def _():
                        drain_scatter(sl, b2)

                    @pl.when(gg + 2 < NB)
                    def _():
                        start_gather(sl, gg + 2, b2)

            drain_scatter(sl, 2)
            drain_scatter(sl, 3)

            @pl.when(blk + 1 < NBLK)
            def _():
                pltpu.make_async_copy(
                    src_hbm.at[wid, blk + 1], src_v.at[nsl], isem).wait()
                pltpu.make_async_copy(
                    dst_hbm.at[wid, blk + 1], dst_v.at[nsl], isem).wait()
                pltpu.make_async_copy(
                    w_hbm.at[wid, blk + 1], w_v.at[nsl], isem).wait()

        plsc.subcore_barrier()

        obuf = rows.at[0, pl.ds(0, ZR)]
        for i in range(ROWS0 // ZR):
            r0 = row0 + i * ZR
            pltpu.sync_copy(acc.at[pl.ds(r0, ZR)], obuf)
            pltpu.sync_copy(obuf, out_hbm.at[c, pl.ds(r0, ZR)])

        @pl.when(s == NS - 1)
        def _():
            tbuf = rows.at[0, pl.ds(0, TAIL)]
            pltpu.sync_copy(acc.at[pl.ds(TAIL0, TAIL)], tbuf)
            pltpu.sync_copy(tbuf, out_hbm.at[c, pl.ds(TAIL0, TAIL)])

    return sc_scatter


_SC_SCATTER = _sc_scatter_fn()

BT = 2000


def _dotT(a, w):
    return lax.dot_general(a, w, (((1,), (1,)), ((), ())),
                           preferred_element_type=_F32)


def _layer_body(p_ref, x_ref, wrel_ref, b_ref, wroot_ref, o_ref):
    agg = p_ref[0] + p_ref[1]
    t = _dotT(agg, wrel_ref[...]) + _dotT(x_ref[...], wroot_ref[...]) + b_ref[...]
    o_ref[...] = jnp.maximum(t, 0.0)


def _tc_layer(p, x, w_rel, b_rel, w_root):
    return pl.pallas_call(
        _layer_body,
        grid=(N // BT,),
        in_specs=[
            pl.BlockSpec((NC, BT, D), lambda i: (0, i, 0)),
            pl.BlockSpec((BT, D), lambda i: (i, 0)),
            pl.BlockSpec((D, D), lambda i: (0, 0)),
            pl.BlockSpec((1, D), lambda i: (0, 0)),
            pl.BlockSpec((D, D), lambda i: (0, 0)),
        ],
        out_specs=pl.BlockSpec((BT, D), lambda i: (i, 0)),
        out_shape=jax.ShapeDtypeStruct((N, D), _F32),
    )(p, x, w_rel, b_rel.reshape(1, D), w_root)


def _final_body(p_ref, h_ref, wrel_ref, b_ref, wroot_ref, wfc_ref, bfc_ref, o_ref):
    agg = p_ref[0] + p_ref[1]
    h2 = jnp.maximum(
        _dotT(agg, wrel_ref[...]) + _dotT(h_ref[...], wroot_ref[...]) + b_ref[...],
        0.0,
    )
    o_ref[...] = _dotT(h2, wfc_ref[...]) + bfc_ref[...]


def _tc_final(p, h, w_rel, b_rel, w_root, wfc, bfc):
    return pl.pallas_call(
        _final_body,
        grid=(N // BT,),
        in_specs=[
            pl.BlockSpec((NC, BT, D), lambda i: (0, i, 0)),
            pl.BlockSpec((BT, D), lambda i: (i, 0)),
            pl.BlockSpec((D, D), lambda i: (0, 0)),
            pl.BlockSpec((1, D), lambda i: (0, 0)),
            pl.BlockSpec((D, D), lambda i: (0, 0)),
            pl.BlockSpec((C, D), lambda i: (0, 0)),
            pl.BlockSpec((1, C), lambda i: (0, 0)),
        ],
        out_specs=pl.BlockSpec((BT, C), lambda i: (i, 0)),
        out_shape=jax.ShapeDtypeStruct((N, C), _F32),
    )(p, h, w_rel, b_rel.reshape(1, D), w_root, wfc, bfc.reshape(1, C))


def kernel(x, edge_index, edge_attr, W1_rel, b1_rel, W1_root,
           W2_rel, b2_rel, W2_root, Wfc, bfc):
    pad = ((0, 0), (0, PAD))
    src = jnp.pad(edge_index[0].reshape(NW, EW), pad).reshape(NW, NBLK, NB, K)
    dst = jnp.pad(edge_index[1].reshape(NW, EW), pad).reshape(NW, NBLK, NB, K)
    w = jnp.pad(edge_attr.reshape(NW, EW), pad).reshape(NW, NBLK, NB * K)

    p1 = _SC_SCATTER(x, src, dst, w)
    h1 = _tc_layer(p1, x, W1_rel, b1_rel, W1_root)
    p2 = _SC_SCATTER(h1, src, dst, w)
    return _tc_final(p2, h1, W2_rel, b2_rel, W2_root, Wfc, bfc)
